# baseline (speedup 1.0000x reference)
import functools

import jax
import jax.numpy as jnp
import numpy as np
from jax import lax
from jax.experimental import pallas as pl
from jax.experimental.pallas import tpu as pltpu
from jax.experimental.pallas import tpu_sc as plsc

_N = 10000
_E = 320000
_D = 128
_C = 10
_L = 4

_NC = 2
_NS = 16
_NW = _NC * _NS
_EPW = _E // _NW
_K = 40
_NCHUNK = _EPW // _K
_SEG = 10
_CPS = _NCHUNK // _SEG
_NBUF = 6
_NP = 10112
_RPT = _NP // _NS

_BN_SCALE = float(1.0 / np.sqrt(1.0 + 1e-5))


def _segsum_body(cur_hbm, src_hbm, dst_hbm, out_hbm, sidx0, didx0, sidx1,
                 didx1, *rest):
    bufs = rest[:_NBUF]
    acc = rest[_NBUF]
    gsems = rest[_NBUF + 1:2 * _NBUF + 1]
    semi0, semi1 = rest[2 * _NBUF + 1:]
    rows0 = bufs[0]
    c = lax.axis_index("c")
    s = lax.axis_index("s")
    w = c * _NS + s
    sidx = (sidx0, sidx1)
    didx = (didx0, didx1)

    f0 = pltpu.async_copy(src_hbm.at[w, 0], sidx0, semi0)
    f1 = pltpu.async_copy(dst_hbm.at[w, 0], didx0, semi1)

    def _zstore(i, carry):
        r = i // (_D // 16)
        col = (i % (_D // 16)) * 16
        rows0[r, pl.ds(col, 16)] = jnp.zeros((16,), jnp.float32)
        return carry

    lax.fori_loop(0, _K * (_D // 16), _zstore, 0)
    for j in range(_RPT // _K):
        pltpu.sync_copy(rows0, acc.at[pl.ds(s * _RPT + j * _K, _K)])
    tail_rows = _RPT - (_RPT // _K) * _K
    if tail_rows:
        pltpu.sync_copy(
            rows0.at[pl.ds(0, tail_rows)],
            acc.at[pl.ds(s * _RPT + (_RPT // _K) * _K, tail_rows)])
    f0.wait()
    f1.wait()
    plsc.subcore_barrier()

    for g in range(_SEG):
        si, di = sidx[g % 2], didx[g % 2]
        if g + 1 < _SEG:
            nf0 = pltpu.async_copy(src_hbm.at[w, g + 1], sidx[(g + 1) % 2],
                                   semi0)
            nf1 = pltpu.async_copy(dst_hbm.at[w, g + 1], didx[(g + 1) % 2],
                                   semi1)

        def _fire(chunk, b):
            return pltpu.async_copy(cur_hbm.at[si.at[chunk]], bufs[b],
                                    gsems[b])

        def _drain(chunk, b):
            pltpu.make_async_copy(cur_hbm.at[si.at[chunk]], bufs[b],
                                  gsems[b]).wait()
            pltpu.sync_copy(bufs[b], acc.at[di.at[chunk]], add=True)

        for j in range(_NBUF - 1):
            _fire(j, j)

        it_max = ((_CPS - 2 * _NBUF + 1) // _NBUF) * _NBUF
        main_end = it_max + _NBUF

        @pl.loop(0, it_max + 1, step=_NBUF)
        def _round(it):
            for b in range(_NBUF):
                _fire(it + b + _NBUF - 1, (b + _NBUF - 1) % _NBUF)
                _drain(it + b, b)

        for chunk in range(main_end, _CPS):
            if chunk + _NBUF - 1 < _CPS:
                _fire(chunk + _NBUF - 1, (chunk + _NBUF - 1) % _NBUF)
            _drain(chunk, chunk % _NBUF)
        if g + 1 < _SEG:
            nf0.wait()
            nf1.wait()
    plsc.subcore_barrier()

    pltpu.sync_copy(acc.at[pl.ds(s * _RPT, _RPT)],
                    out_hbm.at[c, pl.ds(s * _RPT, _RPT)])


@functools.cache
def _segsum_call():
    return pl.kernel(
        _segsum_body,
        out_type=jax.ShapeDtypeStruct((_NC, _NP, _D), jnp.float32),
        mesh=plsc.VectorSubcoreMesh(core_axis_name="c", subcore_axis_name="s",
                                    num_cores=_NC, num_subcores=_NS),
        scratch_types=(
            [pltpu.VMEM((_CPS, _K), jnp.int32)] * 4
            + [pltpu.VMEM((_K, _D), jnp.float32)] * _NBUF
            + [pltpu.VMEM_SHARED((_NP, _D), jnp.float32)]
            + [pltpu.SemaphoreType.DMA] * (_NBUF + 2)
        ),
    )

_R = 400


def _layer0_tc_body(cur_ref, p_ref, w1_ref, b1_ref, g1_ref, be1_ref, w2_ref,
                    b2_ref, gl_ref, bel_ref, pwa_ref, pwb_ref, pb_ref,
                    out_ref, score_ref):
    x = cur_ref[...]
    xin = x + p_ref[0] + p_ref[1]
    a1 = g1_ref[...] * _BN_SCALE
    w1 = w1_ref[...] * a1
    b1 = b1_ref[...] * a1 + be1_ref[...]
    t = jnp.maximum(jnp.dot(xin, w1, preferred_element_type=jnp.float32) + b1,
                    0.0)
    al = gl_ref[...] * _BN_SCALE
    w2 = w2_ref[...] * al
    b2 = b2_ref[...] * al + bel_ref[...]
    u = jnp.maximum(jnp.dot(t, w2, preferred_element_type=jnp.float32) + b2,
                    0.0)
    new = x + u
    out_ref[...] = new
    score_ref[...] = (
        jnp.dot(x, pwa_ref[...], preferred_element_type=jnp.float32)
        + jnp.dot(new, pwb_ref[...], preferred_element_type=jnp.float32)
        + jnp.sum(pb_ref[...], axis=0, keepdims=True))


def _layer_tc_body(cur_ref, p_ref, w1_ref, b1_ref, g1_ref, be1_ref, w2_ref,
                   b2_ref, gl_ref, bel_ref, pwb_ref, sin_ref,
                   out_ref, score_ref):
    x = cur_ref[...]
    xin = x + p_ref[0] + p_ref[1]
    a1 = g1_ref[...] * _BN_SCALE
    w1 = w1_ref[...] * a1
    b1 = b1_ref[...] * a1 + be1_ref[...]
    t = jnp.maximum(jnp.dot(xin, w1, preferred_element_type=jnp.float32) + b1,
                    0.0)
    al = gl_ref[...] * _BN_SCALE
    w2 = w2_ref[...] * al
    b2 = b2_ref[...] * al + bel_ref[...]
    u = jnp.maximum(jnp.dot(t, w2, preferred_element_type=jnp.float32) + b2,
                    0.0)
    new = x + u
    out_ref[...] = new
    score_ref[...] = sin_ref[...] + jnp.dot(
        new, pwb_ref[...], preferred_element_type=jnp.float32)


def _row_spec():
    return pl.BlockSpec((_R, _D), lambda i: (i, 0))


def _full_spec(shape):
    return pl.BlockSpec(shape, lambda i: (0,) * len(shape))


@functools.cache
def _layer_call(first):
    body = _layer0_tc_body if first else _layer_tc_body
    in_specs = [
        _row_spec(),
        pl.BlockSpec((_NC, _R, _D), lambda i: (0, i, 0)),
        _full_spec((_D, _D)),
        _full_spec((1, _D)),
        _full_spec((1, _D)),
        _full_spec((1, _D)),
        _full_spec((_D, _D)),
        _full_spec((1, _D)),
        _full_spec((1, _D)),
        _full_spec((1, _D)),
    ]
    if first:
        in_specs += [
            _full_spec((_D, _D)),
            _full_spec((_D, _D)),
            _full_spec((8, _D)),
        ]
    else:
        in_specs += [
            _full_spec((_D, _D)),
            _row_spec(),
        ]
    return pl.pallas_call(
        body,
        grid=(_N // _R,),
        in_specs=in_specs,
        out_specs=[_row_spec(), _row_spec()],
        out_shape=[
            jax.ShapeDtypeStruct((_N, _D), jnp.float32),
            jax.ShapeDtypeStruct((_N, _D), jnp.float32),
        ],
    )


def kernel(h, edge_index, e, W1, b1, W2, b2, g1, be1, gL, beL, Pw, Pb):
    del e
    src = edge_index[0].reshape(_NW, _SEG, _CPS, _K)
    dst = edge_index[1].reshape(_NW, _SEG, _CPS, _K)
    pwp = jnp.pad(Pw, ((0, 0), (0, 0), (0, _D - _C)))
    pbp = jnp.pad(Pb, ((0, 8 - (_L + 1)), (0, _D - _C)))
    row = lambda v: v.reshape(1, _D)
    cur = h
    score = None
    for i in range(_L):
        parts = _segsum_call()(cur, src, dst)
        common = (cur, parts, W1[i], row(b1[i]), row(g1[i]), row(be1[i]),
                  W2[i], row(b2[i]), row(gL[i]), row(beL[i]))
        if i == 0:
            cur, score = _layer_call(True)(*common, pwp[0], pwp[1], pbp)
        else:
            cur, score = _layer_call(False)(*common, pwp[i + 1], score)
    return score[:, :_C]


# speedup vs baseline: 1.0418x; 1.0418x over previous
import functools

import jax
import jax.numpy as jnp
import numpy as np
from jax import lax
from jax.experimental import pallas as pl
from jax.experimental.pallas import tpu as pltpu
from jax.experimental.pallas import tpu_sc as plsc

_N = 10000
_E = 320000
_D = 128
_C = 10
_L = 4

_NC = 2
_NS = 16
_NW = _NC * _NS
_EPW = _E // _NW
_K = 80
_NCHUNK = _EPW // _K
_SEG = 5
_CPS = _NCHUNK // _SEG
_NBUF = 3
_NP = 10112
_RPT = _NP // _NS

_BN_SCALE = float(1.0 / np.sqrt(1.0 + 1e-5))


def _segsum_body(cur_hbm, src_hbm, dst_hbm, out_hbm, sidx0, didx0, sidx1,
                 didx1, *rest):
    bufs = rest[:_NBUF]
    acc = rest[_NBUF]
    gsems = rest[_NBUF + 1:2 * _NBUF + 1]
    semi0, semi1 = rest[2 * _NBUF + 1:]
    rows0 = bufs[0]
    c = lax.axis_index("c")
    s = lax.axis_index("s")
    w = c * _NS + s
    sidx = (sidx0, sidx1)
    didx = (didx0, didx1)

    f0 = pltpu.async_copy(src_hbm.at[w, 0], sidx0, semi0)
    f1 = pltpu.async_copy(dst_hbm.at[w, 0], didx0, semi1)

    def _zstore(i, carry):
        r = i // (_D // 16)
        col = (i % (_D // 16)) * 16
        rows0[r, pl.ds(col, 16)] = jnp.zeros((16,), jnp.float32)
        return carry

    lax.fori_loop(0, _K * (_D // 16), _zstore, 0)
    for j in range(_RPT // _K):
        pltpu.sync_copy(rows0, acc.at[pl.ds(s * _RPT + j * _K, _K)])
    tail_rows = _RPT - (_RPT // _K) * _K
    if tail_rows:
        pltpu.sync_copy(
            rows0.at[pl.ds(0, tail_rows)],
            acc.at[pl.ds(s * _RPT + (_RPT // _K) * _K, tail_rows)])
    f0.wait()
    f1.wait()
    plsc.subcore_barrier()

    for g in range(_SEG):
        si, di = sidx[g % 2], didx[g % 2]
        if g + 1 < _SEG:
            nf0 = pltpu.async_copy(src_hbm.at[w, g + 1], sidx[(g + 1) % 2],
                                   semi0)
            nf1 = pltpu.async_copy(dst_hbm.at[w, g + 1], didx[(g + 1) % 2],
                                   semi1)

        def _fire(chunk, b):
            return pltpu.async_copy(cur_hbm.at[si.at[chunk]], bufs[b],
                                    gsems[b])

        def _drain(chunk, b):
            pltpu.make_async_copy(cur_hbm.at[si.at[chunk]], bufs[b],
                                  gsems[b]).wait()
            pltpu.sync_copy(bufs[b], acc.at[di.at[chunk]], add=True)

        for j in range(_NBUF - 1):
            _fire(j, j)

        it_max = ((_CPS - 2 * _NBUF + 1) // _NBUF) * _NBUF
        main_end = it_max + _NBUF

        @pl.loop(0, it_max + 1, step=_NBUF)
        def _round(it):
            for b in range(_NBUF):
                _fire(it + b + _NBUF - 1, (b + _NBUF - 1) % _NBUF)
                _drain(it + b, b)

        for chunk in range(main_end, _CPS):
            if chunk + _NBUF - 1 < _CPS:
                _fire(chunk + _NBUF - 1, (chunk + _NBUF - 1) % _NBUF)
            _drain(chunk, chunk % _NBUF)
        if g + 1 < _SEG:
            nf0.wait()
            nf1.wait()
    plsc.subcore_barrier()

    pltpu.sync_copy(acc.at[pl.ds(s * _RPT, _RPT)],
                    out_hbm.at[c, pl.ds(s * _RPT, _RPT)])


@functools.cache
def _segsum_call():
    return pl.kernel(
        _segsum_body,
        out_type=jax.ShapeDtypeStruct((_NC, _NP, _D), jnp.float32),
        mesh=plsc.VectorSubcoreMesh(core_axis_name="c", subcore_axis_name="s",
                                    num_cores=_NC, num_subcores=_NS),
        scratch_types=(
            [pltpu.VMEM((_CPS, _K), jnp.int32)] * 4
            + [pltpu.VMEM((_K, _D), jnp.float32)] * _NBUF
            + [pltpu.VMEM_SHARED((_NP, _D), jnp.float32)]
            + [pltpu.SemaphoreType.DMA] * (_NBUF + 2)
        ),
    )

_R = 400


def _layer0_tc_body(cur_ref, p_ref, w1_ref, b1_ref, g1_ref, be1_ref, w2_ref,
                    b2_ref, gl_ref, bel_ref, pwa_ref, pwb_ref, pb_ref,
                    out_ref, score_ref):
    x = cur_ref[...]
    xin = x + p_ref[0] + p_ref[1]
    a1 = g1_ref[...] * _BN_SCALE
    w1 = w1_ref[...] * a1
    b1 = b1_ref[...] * a1 + be1_ref[...]
    t = jnp.maximum(jnp.dot(xin, w1, preferred_element_type=jnp.float32) + b1,
                    0.0)
    al = gl_ref[...] * _BN_SCALE
    w2 = w2_ref[...] * al
    b2 = b2_ref[...] * al + bel_ref[...]
    u = jnp.maximum(jnp.dot(t, w2, preferred_element_type=jnp.float32) + b2,
                    0.0)
    new = x + u
    out_ref[...] = new
    score_ref[...] = (
        jnp.dot(x, pwa_ref[...], preferred_element_type=jnp.float32)
        + jnp.dot(new, pwb_ref[...], preferred_element_type=jnp.float32)
        + jnp.sum(pb_ref[...], axis=0, keepdims=True))


def _layer_tc_body(cur_ref, p_ref, w1_ref, b1_ref, g1_ref, be1_ref, w2_ref,
                   b2_ref, gl_ref, bel_ref, pwb_ref, sin_ref,
                   out_ref, score_ref):
    x = cur_ref[...]
    xin = x + p_ref[0] + p_ref[1]
    a1 = g1_ref[...] * _BN_SCALE
    w1 = w1_ref[...] * a1
    b1 = b1_ref[...] * a1 + be1_ref[...]
    t = jnp.maximum(jnp.dot(xin, w1, preferred_element_type=jnp.float32) + b1,
                    0.0)
    al = gl_ref[...] * _BN_SCALE
    w2 = w2_ref[...] * al
    b2 = b2_ref[...] * al + bel_ref[...]
    u = jnp.maximum(jnp.dot(t, w2, preferred_element_type=jnp.float32) + b2,
                    0.0)
    new = x + u
    out_ref[...] = new
    score_ref[...] = sin_ref[...] + jnp.dot(
        new, pwb_ref[...], preferred_element_type=jnp.float32)


def _row_spec():
    return pl.BlockSpec((_R, _D), lambda i: (i, 0))


def _full_spec(shape):
    return pl.BlockSpec(shape, lambda i: (0,) * len(shape))


@functools.cache
def _layer_call(first):
    body = _layer0_tc_body if first else _layer_tc_body
    in_specs = [
        _row_spec(),
        pl.BlockSpec((_NC, _R, _D), lambda i: (0, i, 0)),
        _full_spec((_D, _D)),
        _full_spec((1, _D)),
        _full_spec((1, _D)),
        _full_spec((1, _D)),
        _full_spec((_D, _D)),
        _full_spec((1, _D)),
        _full_spec((1, _D)),
        _full_spec((1, _D)),
    ]
    if first:
        in_specs += [
            _full_spec((_D, _D)),
            _full_spec((_D, _D)),
            _full_spec((8, _D)),
        ]
    else:
        in_specs += [
            _full_spec((_D, _D)),
            _row_spec(),
        ]
    return pl.pallas_call(
        body,
        grid=(_N // _R,),
        in_specs=in_specs,
        out_specs=[_row_spec(), _row_spec()],
        out_shape=[
            jax.ShapeDtypeStruct((_N, _D), jnp.float32),
            jax.ShapeDtypeStruct((_N, _D), jnp.float32),
        ],
    )


def kernel(h, edge_index, e, W1, b1, W2, b2, g1, be1, gL, beL, Pw, Pb):
    del e
    src = edge_index[0].reshape(_NW, _SEG, _CPS, _K)
    dst = edge_index[1].reshape(_NW, _SEG, _CPS, _K)
    pwp = jnp.pad(Pw, ((0, 0), (0, 0), (0, _D - _C)))
    pbp = jnp.pad(Pb, ((0, 8 - (_L + 1)), (0, _D - _C)))
    row = lambda v: v.reshape(1, _D)
    cur = h
    score = None
    for i in range(_L):
        parts = _segsum_call()(cur, src, dst)
        common = (cur, parts, W1[i], row(b1[i]), row(g1[i]), row(be1[i]),
                  W2[i], row(b2[i]), row(gL[i]), row(beL[i]))
        if i == 0:
            cur, score = _layer_call(True)(*common, pwp[0], pwp[1], pbp)
        else:
            cur, score = _layer_call(False)(*common, pwp[i + 1], score)
    return score[:, :_C]


# speedup vs baseline: 1.1181x; 1.0732x over previous
import functools

import jax
import jax.numpy as jnp
import numpy as np
from jax import lax
from jax.experimental import pallas as pl
from jax.experimental.pallas import tpu as pltpu
from jax.experimental.pallas import tpu_sc as plsc

_N = 10000
_E = 320000
_D = 128
_C = 10
_L = 4

_NC = 2
_NS = 16
_NW = _NC * _NS
_EPW = _E // _NW
_K = 80
_NCHUNK = _EPW // _K
_SEG = 5
_CPS = _NCHUNK // _SEG
_NBUF = 3
_NP = 10112
_RPT = _NP // _NS

_BN_SCALE = float(1.0 / np.sqrt(1.0 + 1e-5))


def _segsum_body(cur_hbm, src_hbm, dst_hbm, out_hbm, sidx0, didx0, sidx1,
                 didx1, *rest):
    bufs = rest[:_NBUF]
    acc = rest[_NBUF]
    gsems = rest[_NBUF + 1:2 * _NBUF + 1]
    semi0, semi1 = rest[2 * _NBUF + 1:]
    rows0 = bufs[0]
    c = lax.axis_index("c")
    s = lax.axis_index("s")
    w = c * _NS + s
    sidx = (sidx0, sidx1)
    didx = (didx0, didx1)

    f0 = pltpu.async_copy(src_hbm.at[w, 0], sidx0, semi0)
    f1 = pltpu.async_copy(dst_hbm.at[w, 0], didx0, semi1)

    def _zstore(i, carry):
        r = i // (_D // 16)
        col = (i % (_D // 16)) * 16
        rows0[r, pl.ds(col, 16)] = jnp.zeros((16,), jnp.float32)
        return carry

    lax.fori_loop(0, _K * (_D // 16), _zstore, 0)
    for j in range(_RPT // _K):
        pltpu.sync_copy(rows0, acc.at[pl.ds(s * _RPT + j * _K, _K)])
    tail_rows = _RPT - (_RPT // _K) * _K
    if tail_rows:
        pltpu.sync_copy(
            rows0.at[pl.ds(0, tail_rows)],
            acc.at[pl.ds(s * _RPT + (_RPT // _K) * _K, tail_rows)])
    f0.wait()
    f1.wait()
    plsc.subcore_barrier()

    for g in range(_SEG):
        si, di = sidx[g % 2], didx[g % 2]
        if g + 1 < _SEG:
            nf0 = pltpu.async_copy(src_hbm.at[w, g + 1], sidx[(g + 1) % 2],
                                   semi0)
            nf1 = pltpu.async_copy(dst_hbm.at[w, g + 1], didx[(g + 1) % 2],
                                   semi1)

        def _fire(chunk, b):
            return pltpu.async_copy(cur_hbm.at[si.at[chunk]], bufs[b],
                                    gsems[b])

        def _drain(chunk, b):
            pltpu.make_async_copy(cur_hbm.at[si.at[chunk]], bufs[b],
                                  gsems[b]).wait()
            pltpu.sync_copy(bufs[b], acc.at[di.at[chunk]], add=True)

        for j in range(_NBUF - 1):
            _fire(j, j)

        it_max = ((_CPS - 2 * _NBUF + 1) // _NBUF) * _NBUF
        main_end = it_max + _NBUF

        @pl.loop(0, it_max + 1, step=_NBUF)
        def _round(it):
            for b in range(_NBUF):
                _fire(it + b + _NBUF - 1, (b + _NBUF - 1) % _NBUF)
                _drain(it + b, b)

        for chunk in range(main_end, _CPS):
            if chunk + _NBUF - 1 < _CPS:
                _fire(chunk + _NBUF - 1, (chunk + _NBUF - 1) % _NBUF)
            _drain(chunk, chunk % _NBUF)
        if g + 1 < _SEG:
            nf0.wait()
            nf1.wait()
    plsc.subcore_barrier()

    pltpu.sync_copy(acc.at[pl.ds(s * _RPT, _RPT)],
                    out_hbm.at[c, pl.ds(s * _RPT, _RPT)])


@functools.cache
def _segsum_call():
    return pl.kernel(
        _segsum_body,
        out_type=jax.ShapeDtypeStruct((_NC, _NP, _D), jnp.float32),
        mesh=plsc.VectorSubcoreMesh(core_axis_name="c", subcore_axis_name="s",
                                    num_cores=_NC, num_subcores=_NS),
        scratch_types=(
            [pltpu.VMEM((_CPS, _K), jnp.int32)] * 4
            + [pltpu.VMEM((_K, _D), jnp.float32)] * _NBUF
            + [pltpu.VMEM_SHARED((_NP, _D), jnp.float32)]
            + [pltpu.SemaphoreType.DMA] * (_NBUF + 2)
        ),
    )

_R = 1000


def _layer0_tc_body(cur_ref, p_ref, w1_ref, b1_ref, g1_ref, be1_ref, w2_ref,
                    b2_ref, gl_ref, bel_ref, pwa_ref, pwb_ref, pb_ref,
                    out_ref, score_ref):
    x = cur_ref[...]
    xin = x + p_ref[0] + p_ref[1]
    a1 = g1_ref[...] * _BN_SCALE
    w1 = w1_ref[...] * a1
    b1 = b1_ref[...] * a1 + be1_ref[...]
    t = jnp.maximum(jnp.dot(xin, w1, preferred_element_type=jnp.float32) + b1,
                    0.0)
    al = gl_ref[...] * _BN_SCALE
    w2 = w2_ref[...] * al
    b2 = b2_ref[...] * al + bel_ref[...]
    u = jnp.maximum(jnp.dot(t, w2, preferred_element_type=jnp.float32) + b2,
                    0.0)
    new = x + u
    out_ref[...] = new
    score_ref[...] = (
        jnp.dot(x, pwa_ref[...], preferred_element_type=jnp.float32)
        + jnp.dot(new, pwb_ref[...], preferred_element_type=jnp.float32)
        + jnp.sum(pb_ref[...], axis=0, keepdims=True))


def _layer_tc_body(cur_ref, p_ref, w1_ref, b1_ref, g1_ref, be1_ref, w2_ref,
                   b2_ref, gl_ref, bel_ref, pwb_ref, sin_ref,
                   out_ref, score_ref):
    x = cur_ref[...]
    xin = x + p_ref[0] + p_ref[1]
    a1 = g1_ref[...] * _BN_SCALE
    w1 = w1_ref[...] * a1
    b1 = b1_ref[...] * a1 + be1_ref[...]
    t = jnp.maximum(jnp.dot(xin, w1, preferred_element_type=jnp.float32) + b1,
                    0.0)
    al = gl_ref[...] * _BN_SCALE
    w2 = w2_ref[...] * al
    b2 = b2_ref[...] * al + bel_ref[...]
    u = jnp.maximum(jnp.dot(t, w2, preferred_element_type=jnp.float32) + b2,
                    0.0)
    new = x + u
    out_ref[...] = new
    score_ref[...] = sin_ref[...] + jnp.dot(
        new, pwb_ref[...], preferred_element_type=jnp.float32)


def _row_spec():
    return pl.BlockSpec((_R, _D), lambda i: (i, 0))


def _full_spec(shape):
    return pl.BlockSpec(shape, lambda i: (0,) * len(shape))


@functools.cache
def _layer_call(first):
    body = _layer0_tc_body if first else _layer_tc_body
    in_specs = [
        _row_spec(),
        pl.BlockSpec((_NC, _R, _D), lambda i: (0, i, 0)),
        _full_spec((_D, _D)),
        _full_spec((1, _D)),
        _full_spec((1, _D)),
        _full_spec((1, _D)),
        _full_spec((_D, _D)),
        _full_spec((1, _D)),
        _full_spec((1, _D)),
        _full_spec((1, _D)),
    ]
    if first:
        in_specs += [
            _full_spec((_D, _D)),
            _full_spec((_D, _D)),
            _full_spec((8, _D)),
        ]
    else:
        in_specs += [
            _full_spec((_D, _D)),
            _row_spec(),
        ]
    return pl.pallas_call(
        body,
        grid=(_N // _R,),
        in_specs=in_specs,
        out_specs=[_row_spec(), _row_spec()],
        out_shape=[
            jax.ShapeDtypeStruct((_N, _D), jnp.float32),
            jax.ShapeDtypeStruct((_N, _D), jnp.float32),
        ],
    )


def kernel(h, edge_index, e, W1, b1, W2, b2, g1, be1, gL, beL, Pw, Pb):
    del e
    src = edge_index[0].reshape(_NW, _SEG, _CPS, _K)
    dst = edge_index[1].reshape(_NW, _SEG, _CPS, _K)
    pwp = jnp.pad(Pw, ((0, 0), (0, 0), (0, _D - _C)))
    pbp = jnp.pad(Pb, ((0, 8 - (_L + 1)), (0, _D - _C)))
    row = lambda v: v.reshape(1, _D)
    cur = h
    score = None
    for i in range(_L):
        parts = _segsum_call()(cur, src, dst)
        common = (cur, parts, W1[i], row(b1[i]), row(g1[i]), row(be1[i]),
                  W2[i], row(b2[i]), row(gL[i]), row(beL[i]))
        if i == 0:
            cur, score = _layer_call(True)(*common, pwp[0], pwp[1], pbp)
        else:
            cur, score = _layer_call(False)(*common, pwp[i + 1], score)
    return score[:, :_C]


# speedup vs baseline: 1.1420x; 1.0214x over previous
import functools

import jax
import jax.numpy as jnp
import numpy as np
from jax import lax
from jax.experimental import pallas as pl
from jax.experimental.pallas import tpu as pltpu
from jax.experimental.pallas import tpu_sc as plsc

_N = 10000
_E = 320000
_D = 128
_C = 10
_L = 4

_NC = 2
_NS = 16
_NW = _NC * _NS
_EPW = _E // _NW
_K = 80
_NCHUNK = _EPW // _K
_SEG = 5
_CPS = _NCHUNK // _SEG
_NBUF = 3
_NP = 10112
_RPT = _NP // _NS

_BN_SCALE = float(1.0 / np.sqrt(1.0 + 1e-5))


def _segsum_body(cur_hbm, src_hbm, dst_hbm, out_hbm, sidx0, didx0, sidx1,
                 didx1, *rest):
    bufs = rest[:_NBUF]
    acc = rest[_NBUF]
    gsems = rest[_NBUF + 1:2 * _NBUF + 1]
    semi0, semi1 = rest[2 * _NBUF + 1:]
    rows0 = bufs[0]
    c = lax.axis_index("c")
    s = lax.axis_index("s")
    w = c * _NS + s
    sidx = (sidx0, sidx1)
    didx = (didx0, didx1)

    f0 = pltpu.async_copy(src_hbm.at[w, 0], sidx0, semi0)
    f1 = pltpu.async_copy(dst_hbm.at[w, 0], didx0, semi1)

    def _zstore(i, carry):
        r = i // (_D // 16)
        col = (i % (_D // 16)) * 16
        rows0[r, pl.ds(col, 16)] = jnp.zeros((16,), jnp.float32)
        return carry

    lax.fori_loop(0, _K * (_D // 16), _zstore, 0)
    for j in range(_RPT // _K):
        pltpu.sync_copy(rows0, acc.at[pl.ds(s * _RPT + j * _K, _K)])
    tail_rows = _RPT - (_RPT // _K) * _K
    if tail_rows:
        pltpu.sync_copy(
            rows0.at[pl.ds(0, tail_rows)],
            acc.at[pl.ds(s * _RPT + (_RPT // _K) * _K, tail_rows)])
    f0.wait()
    f1.wait()
    plsc.subcore_barrier()

    for g in range(_SEG):
        si, di = sidx[g % 2], didx[g % 2]
        if g + 1 < _SEG:
            nf0 = pltpu.async_copy(src_hbm.at[w, g + 1], sidx[(g + 1) % 2],
                                   semi0)
            nf1 = pltpu.async_copy(dst_hbm.at[w, g + 1], didx[(g + 1) % 2],
                                   semi1)

        def _fire(chunk, b):
            return pltpu.async_copy(cur_hbm.at[si.at[chunk]], bufs[b],
                                    gsems[b])

        def _drain(chunk, b):
            pltpu.make_async_copy(cur_hbm.at[si.at[chunk]], bufs[b],
                                  gsems[b]).wait()
            pltpu.sync_copy(bufs[b], acc.at[di.at[chunk]], add=True)

        for j in range(_NBUF - 1):
            _fire(j, j)

        it_max = ((_CPS - 2 * _NBUF + 1) // _NBUF) * _NBUF
        main_end = it_max + _NBUF

        @pl.loop(0, it_max + 1, step=_NBUF)
        def _round(it):
            for b in range(_NBUF):
                _fire(it + b + _NBUF - 1, (b + _NBUF - 1) % _NBUF)
                _drain(it + b, b)

        for chunk in range(main_end, _CPS):
            if chunk + _NBUF - 1 < _CPS:
                _fire(chunk + _NBUF - 1, (chunk + _NBUF - 1) % _NBUF)
            _drain(chunk, chunk % _NBUF)
        if g + 1 < _SEG:
            nf0.wait()
            nf1.wait()
    plsc.subcore_barrier()

    pltpu.sync_copy(acc.at[pl.ds(s * _RPT, _RPT)],
                    out_hbm.at[c, pl.ds(s * _RPT, _RPT)])


@functools.cache
def _segsum_call():
    return pl.kernel(
        _segsum_body,
        out_type=jax.ShapeDtypeStruct((_NC, _NP, _D), jnp.float32),
        mesh=plsc.VectorSubcoreMesh(core_axis_name="c", subcore_axis_name="s",
                                    num_cores=_NC, num_subcores=_NS),
        scratch_types=(
            [pltpu.VMEM((_CPS, _K), jnp.int32)] * 4
            + [pltpu.VMEM((_K, _D), jnp.float32)] * _NBUF
            + [pltpu.VMEM_SHARED((_NP, _D), jnp.float32)]
            + [pltpu.SemaphoreType.DMA] * (_NBUF + 2)
        ),
    )

_R = 2000


def _layer0_tc_body(cur_ref, p_ref, w1_ref, b1_ref, g1_ref, be1_ref, w2_ref,
                    b2_ref, gl_ref, bel_ref, pwa_ref, pwb_ref, pb_ref,
                    out_ref, score_ref):
    x = cur_ref[...]
    xin = x + p_ref[0] + p_ref[1]
    a1 = g1_ref[...] * _BN_SCALE
    w1 = w1_ref[...] * a1
    b1 = b1_ref[...] * a1 + be1_ref[...]
    t = jnp.maximum(jnp.dot(xin, w1, preferred_element_type=jnp.float32) + b1,
                    0.0)
    al = gl_ref[...] * _BN_SCALE
    w2 = w2_ref[...] * al
    b2 = b2_ref[...] * al + bel_ref[...]
    u = jnp.maximum(jnp.dot(t, w2, preferred_element_type=jnp.float32) + b2,
                    0.0)
    new = x + u
    out_ref[...] = new
    score_ref[...] = (
        jnp.dot(x, pwa_ref[...], preferred_element_type=jnp.float32)
        + jnp.dot(new, pwb_ref[...], preferred_element_type=jnp.float32)
        + jnp.sum(pb_ref[...], axis=0, keepdims=True))


def _layer_tc_body(cur_ref, p_ref, w1_ref, b1_ref, g1_ref, be1_ref, w2_ref,
                   b2_ref, gl_ref, bel_ref, pwb_ref, sin_ref,
                   out_ref, score_ref):
    x = cur_ref[...]
    xin = x + p_ref[0] + p_ref[1]
    a1 = g1_ref[...] * _BN_SCALE
    w1 = w1_ref[...] * a1
    b1 = b1_ref[...] * a1 + be1_ref[...]
    t = jnp.maximum(jnp.dot(xin, w1, preferred_element_type=jnp.float32) + b1,
                    0.0)
    al = gl_ref[...] * _BN_SCALE
    w2 = w2_ref[...] * al
    b2 = b2_ref[...] * al + bel_ref[...]
    u = jnp.maximum(jnp.dot(t, w2, preferred_element_type=jnp.float32) + b2,
                    0.0)
    new = x + u
    out_ref[...] = new
    score_ref[...] = sin_ref[...] + jnp.dot(
        new, pwb_ref[...], preferred_element_type=jnp.float32)


def _row_spec():
    return pl.BlockSpec((_R, _D), lambda i: (i, 0))


def _full_spec(shape):
    return pl.BlockSpec(shape, lambda i: (0,) * len(shape))


@functools.cache
def _layer_call(first):
    body = _layer0_tc_body if first else _layer_tc_body
    in_specs = [
        _row_spec(),
        pl.BlockSpec((_NC, _R, _D), lambda i: (0, i, 0)),
        _full_spec((_D, _D)),
        _full_spec((1, _D)),
        _full_spec((1, _D)),
        _full_spec((1, _D)),
        _full_spec((_D, _D)),
        _full_spec((1, _D)),
        _full_spec((1, _D)),
        _full_spec((1, _D)),
    ]
    if first:
        in_specs += [
            _full_spec((_D, _D)),
            _full_spec((_D, _D)),
            _full_spec((8, _D)),
        ]
    else:
        in_specs += [
            _full_spec((_D, _D)),
            _row_spec(),
        ]
    return pl.pallas_call(
        body,
        grid=(_N // _R,),
        in_specs=in_specs,
        out_specs=[_row_spec(), _row_spec()],
        out_shape=[
            jax.ShapeDtypeStruct((_N, _D), jnp.float32),
            jax.ShapeDtypeStruct((_N, _D), jnp.float32),
        ],
    )


def kernel(h, edge_index, e, W1, b1, W2, b2, g1, be1, gL, beL, Pw, Pb):
    del e
    src = edge_index[0].reshape(_NW, _SEG, _CPS, _K)
    dst = edge_index[1].reshape(_NW, _SEG, _CPS, _K)
    pwp = jnp.pad(Pw, ((0, 0), (0, 0), (0, _D - _C)))
    pbp = jnp.pad(Pb, ((0, 8 - (_L + 1)), (0, _D - _C)))
    row = lambda v: v.reshape(1, _D)
    cur = h
    score = None
    for i in range(_L):
        parts = _segsum_call()(cur, src, dst)
        common = (cur, parts, W1[i], row(b1[i]), row(g1[i]), row(be1[i]),
                  W2[i], row(b2[i]), row(gL[i]), row(beL[i]))
        if i == 0:
            cur, score = _layer_call(True)(*common, pwp[0], pwp[1], pbp)
        else:
            cur, score = _layer_call(False)(*common, pwp[i + 1], score)
    return score[:, :_C]


# speedup vs baseline: 1.1527x; 1.0093x over previous
import functools

import jax
import jax.numpy as jnp
import numpy as np
from jax import lax
from jax.experimental import pallas as pl
from jax.experimental.pallas import tpu as pltpu
from jax.experimental.pallas import tpu_sc as plsc

_N = 10000
_E = 320000
_D = 128
_C = 10
_L = 4

_NC = 2
_NS = 16
_NW = _NC * _NS
_EPW = _E // _NW
_K = 80
_NCHUNK = _EPW // _K
_SEG = 5
_CPS = _NCHUNK // _SEG
_NBUF = 3
_NP = 10112
_RPT = _NP // _NS

_BN_SCALE = float(1.0 / np.sqrt(1.0 + 1e-5))


def _segsum_body(cur_hbm, src_hbm, dst_hbm, out_hbm, sidx0, didx0, sidx1,
                 didx1, *rest):
    bufs = rest[:_NBUF]
    acc = rest[_NBUF]
    gsems = rest[_NBUF + 1:2 * _NBUF + 1]
    semi0, semi1 = rest[2 * _NBUF + 1:]
    rows0 = bufs[0]
    c = lax.axis_index("c")
    s = lax.axis_index("s")
    w = c * _NS + s
    sidx = (sidx0, sidx1)
    didx = (didx0, didx1)

    f0 = pltpu.async_copy(src_hbm.at[w, 0], sidx0, semi0)
    f1 = pltpu.async_copy(dst_hbm.at[w, 0], didx0, semi1)

    def _zstore(i, carry):
        r = i // (_D // 16)
        col = (i % (_D // 16)) * 16
        rows0[r, pl.ds(col, 16)] = jnp.zeros((16,), jnp.float32)
        return carry

    lax.fori_loop(0, _K * (_D // 16), _zstore, 0)
    for j in range(_RPT // _K):
        pltpu.sync_copy(rows0, acc.at[pl.ds(s * _RPT + j * _K, _K)])
    tail_rows = _RPT - (_RPT // _K) * _K
    if tail_rows:
        pltpu.sync_copy(
            rows0.at[pl.ds(0, tail_rows)],
            acc.at[pl.ds(s * _RPT + (_RPT // _K) * _K, tail_rows)])
    f0.wait()
    f1.wait()
    plsc.subcore_barrier()

    for g in range(_SEG):
        si, di = sidx[g % 2], didx[g % 2]
        if g + 1 < _SEG:
            nf0 = pltpu.async_copy(src_hbm.at[w, g + 1], sidx[(g + 1) % 2],
                                   semi0)
            nf1 = pltpu.async_copy(dst_hbm.at[w, g + 1], didx[(g + 1) % 2],
                                   semi1)

        def _fire(chunk, b):
            return pltpu.async_copy(cur_hbm.at[si.at[chunk]], bufs[b],
                                    gsems[b])

        def _drain(chunk, b):
            pltpu.make_async_copy(cur_hbm.at[si.at[chunk]], bufs[b],
                                  gsems[b]).wait()
            pltpu.sync_copy(bufs[b], acc.at[di.at[chunk]], add=True)

        for j in range(_NBUF - 1):
            _fire(j, j)

        it_max = ((_CPS - 2 * _NBUF + 1) // _NBUF) * _NBUF
        main_end = it_max + _NBUF

        @pl.loop(0, it_max + 1, step=_NBUF)
        def _round(it):
            for b in range(_NBUF):
                _fire(it + b + _NBUF - 1, (b + _NBUF - 1) % _NBUF)
                _drain(it + b, b)

        for chunk in range(main_end, _CPS):
            if chunk + _NBUF - 1 < _CPS:
                _fire(chunk + _NBUF - 1, (chunk + _NBUF - 1) % _NBUF)
            _drain(chunk, chunk % _NBUF)
        if g + 1 < _SEG:
            nf0.wait()
            nf1.wait()
    plsc.subcore_barrier()

    pltpu.sync_copy(acc.at[pl.ds(s * _RPT, _RPT)],
                    out_hbm.at[c, pl.ds(s * _RPT, _RPT)])


@functools.cache
def _segsum_call():
    return pl.kernel(
        _segsum_body,
        out_type=jax.ShapeDtypeStruct((_NC, _NP, _D), jnp.float32),
        mesh=plsc.VectorSubcoreMesh(core_axis_name="c", subcore_axis_name="s",
                                    num_cores=_NC, num_subcores=_NS),
        scratch_types=(
            [pltpu.VMEM((_CPS, _K), jnp.int32)] * 4
            + [pltpu.VMEM((_K, _D), jnp.float32)] * _NBUF
            + [pltpu.VMEM_SHARED((_NP, _D), jnp.float32)]
            + [pltpu.SemaphoreType.DMA] * (_NBUF + 2)
        ),
    )

_R = 5000


def _layer0_tc_body(cur_ref, p_ref, w1_ref, b1_ref, g1_ref, be1_ref, w2_ref,
                    b2_ref, gl_ref, bel_ref, pwa_ref, pwb_ref, pb_ref,
                    out_ref, score_ref):
    x = cur_ref[...]
    xin = x + p_ref[0] + p_ref[1]
    a1 = g1_ref[...] * _BN_SCALE
    w1 = w1_ref[...] * a1
    b1 = b1_ref[...] * a1 + be1_ref[...]
    t = jnp.maximum(jnp.dot(xin, w1, preferred_element_type=jnp.float32) + b1,
                    0.0)
    al = gl_ref[...] * _BN_SCALE
    w2 = w2_ref[...] * al
    b2 = b2_ref[...] * al + bel_ref[...]
    u = jnp.maximum(jnp.dot(t, w2, preferred_element_type=jnp.float32) + b2,
                    0.0)
    new = x + u
    out_ref[...] = new
    score_ref[...] = (
        jnp.dot(x, pwa_ref[...], preferred_element_type=jnp.float32)
        + jnp.dot(new, pwb_ref[...], preferred_element_type=jnp.float32)
        + jnp.sum(pb_ref[...], axis=0, keepdims=True))


def _layer_tc_body(cur_ref, p_ref, w1_ref, b1_ref, g1_ref, be1_ref, w2_ref,
                   b2_ref, gl_ref, bel_ref, pwb_ref, sin_ref,
                   out_ref, score_ref):
    x = cur_ref[...]
    xin = x + p_ref[0] + p_ref[1]
    a1 = g1_ref[...] * _BN_SCALE
    w1 = w1_ref[...] * a1
    b1 = b1_ref[...] * a1 + be1_ref[...]
    t = jnp.maximum(jnp.dot(xin, w1, preferred_element_type=jnp.float32) + b1,
                    0.0)
    al = gl_ref[...] * _BN_SCALE
    w2 = w2_ref[...] * al
    b2 = b2_ref[...] * al + bel_ref[...]
    u = jnp.maximum(jnp.dot(t, w2, preferred_element_type=jnp.float32) + b2,
                    0.0)
    new = x + u
    out_ref[...] = new
    score_ref[...] = sin_ref[...] + jnp.dot(
        new, pwb_ref[...], preferred_element_type=jnp.float32)


def _row_spec():
    return pl.BlockSpec((_R, _D), lambda i: (i, 0))


def _full_spec(shape):
    return pl.BlockSpec(shape, lambda i: (0,) * len(shape))


@functools.cache
def _layer_call(first):
    body = _layer0_tc_body if first else _layer_tc_body
    in_specs = [
        _row_spec(),
        pl.BlockSpec((_NC, _R, _D), lambda i: (0, i, 0)),
        _full_spec((_D, _D)),
        _full_spec((1, _D)),
        _full_spec((1, _D)),
        _full_spec((1, _D)),
        _full_spec((_D, _D)),
        _full_spec((1, _D)),
        _full_spec((1, _D)),
        _full_spec((1, _D)),
    ]
    if first:
        in_specs += [
            _full_spec((_D, _D)),
            _full_spec((_D, _D)),
            _full_spec((8, _D)),
        ]
    else:
        in_specs += [
            _full_spec((_D, _D)),
            _row_spec(),
        ]
    return pl.pallas_call(
        body,
        grid=(_N // _R,),
        in_specs=in_specs,
        out_specs=[_row_spec(), _row_spec()],
        out_shape=[
            jax.ShapeDtypeStruct((_N, _D), jnp.float32),
            jax.ShapeDtypeStruct((_N, _D), jnp.float32),
        ],
    )


def kernel(h, edge_index, e, W1, b1, W2, b2, g1, be1, gL, beL, Pw, Pb):
    del e
    src = edge_index[0].reshape(_NW, _SEG, _CPS, _K)
    dst = edge_index[1].reshape(_NW, _SEG, _CPS, _K)
    pwp = jnp.pad(Pw, ((0, 0), (0, 0), (0, _D - _C)))
    pbp = jnp.pad(Pb, ((0, 8 - (_L + 1)), (0, _D - _C)))
    row = lambda v: v.reshape(1, _D)
    cur = h
    score = None
    for i in range(_L):
        parts = _segsum_call()(cur, src, dst)
        common = (cur, parts, W1[i], row(b1[i]), row(g1[i]), row(be1[i]),
                  W2[i], row(b2[i]), row(gL[i]), row(beL[i]))
        if i == 0:
            cur, score = _layer_call(True)(*common, pwp[0], pwp[1], pbp)
        else:
            cur, score = _layer_call(False)(*common, pwp[i + 1], score)
    return score[:, :_C]
